# baseline (device time: 304821 ns/iter reference)
import jax
import jax.numpy as jnp
from jax import lax
from jax.experimental import pallas as pl
from jax.experimental.pallas import tpu as pltpu

N_DEV = 4

DIRECT_ROWS = 3072
REV_ROWS = 1024

N_CHUNKS = 4
CHUNK_ROWS = DIRECT_ROWS // N_CHUNKS


def kernel(x, pi):
    def body(
        x_ref,
        pi_ref,
        out_ref,
        t1,
        t2,
        vb,
        copy_sems,
        d_send,
        d_recv,
        s1_send,
        s1_recv,
        s2_send,
        s2_recv,
        s3_send,
        s3_recv,
    ):
        my = lax.axis_index("i")
        dst = pi_ref[my]
        s = lax.rem(dst - my + N_DEV, N_DEV)

        @pl.when(s == 2)
        def _():
            rdma = pltpu.make_async_remote_copy(
                src_ref=x_ref,
                dst_ref=out_ref,
                send_sem=d_send,
                recv_sem=d_recv,
                device_id=dst,
                device_id_type=pl.DeviceIdType.LOGICAL,
            )
            rdma.start()
            rdma.wait()

        @pl.when(s != 2)
        def _():
            rev = lax.rem(my + N_DEV - s, N_DEV)

            direct = pltpu.make_async_remote_copy(
                src_ref=vb,
                dst_ref=out_ref.at[:, pl.ds(0, DIRECT_ROWS), :],
                send_sem=d_send,
                recv_sem=d_recv,
                device_id=dst,
                device_id_type=pl.DeviceIdType.LOGICAL,
            )

            def chunk_copy(c):
                return pltpu.make_async_copy(
                    x_ref.at[:, pl.ds(c * CHUNK_ROWS, CHUNK_ROWS), :],
                    vb.at[:, pl.ds(c * CHUNK_ROWS, CHUNK_ROWS), :],
                    copy_sems.at[c],
                )

            copies = [chunk_copy(c) for c in range(N_CHUNKS)]
            copies[0].start()

            st1 = pltpu.make_async_remote_copy(
                src_ref=x_ref.at[:, pl.ds(DIRECT_ROWS, REV_ROWS), :],
                dst_ref=t1,
                send_sem=s1_send,
                recv_sem=s1_recv,
                device_id=rev,
                device_id_type=pl.DeviceIdType.LOGICAL,
            )
            st1.start()

            for c in range(N_CHUNKS):
                copies[c].wait()
                if c + 1 < N_CHUNKS:
                    copies[c + 1].start()
                pltpu.make_async_remote_copy(
                    src_ref=vb.at[:, pl.ds(c * CHUNK_ROWS, CHUNK_ROWS), :],
                    dst_ref=out_ref.at[
                        :, pl.ds(c * CHUNK_ROWS, CHUNK_ROWS), :
                    ],
                    send_sem=d_send,
                    recv_sem=d_recv,
                    device_id=dst,
                    device_id_type=pl.DeviceIdType.LOGICAL,
                ).start()

            st1.wait_recv()

            st2 = pltpu.make_async_remote_copy(
                src_ref=t1,
                dst_ref=t2,
                send_sem=s2_send,
                recv_sem=s2_recv,
                device_id=rev,
                device_id_type=pl.DeviceIdType.LOGICAL,
            )
            st2.start()
            st2.wait_recv()

            st3 = pltpu.make_async_remote_copy(
                src_ref=t2,
                dst_ref=out_ref.at[:, pl.ds(DIRECT_ROWS, REV_ROWS), :],
                send_sem=s3_send,
                recv_sem=s3_recv,
                device_id=rev,
                device_id_type=pl.DeviceIdType.LOGICAL,
            )
            st3.start()
            st3.wait_recv()

            st1.wait_send()
            st2.wait_send()
            st3.wait_send()
            direct.wait_send()
            direct.wait_recv()

    return pl.pallas_call(
        body,
        out_shape=jax.ShapeDtypeStruct(x.shape, x.dtype),
        in_specs=[
            pl.BlockSpec(memory_space=pl.ANY),
            pl.BlockSpec(memory_space=pltpu.SMEM),
        ],
        out_specs=pl.BlockSpec(memory_space=pl.ANY),
        scratch_shapes=[
            pltpu.VMEM((1, REV_ROWS, 2048), jnp.float32),
            pltpu.VMEM((1, REV_ROWS, 2048), jnp.float32),
            pltpu.VMEM((1, DIRECT_ROWS, 2048), jnp.float32),
            pltpu.SemaphoreType.DMA((N_CHUNKS,)),
            pltpu.SemaphoreType.DMA,
            pltpu.SemaphoreType.DMA,
            pltpu.SemaphoreType.DMA,
            pltpu.SemaphoreType.DMA,
            pltpu.SemaphoreType.DMA,
            pltpu.SemaphoreType.DMA,
            pltpu.SemaphoreType.DMA,
            pltpu.SemaphoreType.DMA,
        ],
        compiler_params=pltpu.CompilerParams(
            has_side_effects=True,
            vmem_limit_bytes=64 * 1024 * 1024,
        ),
    )(x, pi)


# device time: 298776 ns/iter; 1.0202x vs baseline; 1.0202x over previous
import jax
import jax.numpy as jnp
from jax import lax
from jax.experimental import pallas as pl
from jax.experimental.pallas import tpu as pltpu

N_DEV = 4

DIRECT_ROWS = 3072
REV_ROWS = 1024
N_STAGES = 3
N_SUB = 2
SUB_ROWS = REV_ROWS // N_SUB


def kernel(x, pi):
    def body(
        x_ref,
        pi_ref,
        out_ref,
        t1,
        t2,
        d_send,
        d_recv,
        r_send,
        r_recv,
    ):
        my = lax.axis_index("i")
        dst = pi_ref[my]
        s = lax.rem(dst - my + N_DEV, N_DEV)
        barrier = pltpu.get_barrier_semaphore()

        @pl.when(s == 2)
        def _():
            pl.semaphore_signal(
                barrier,
                inc=1,
                device_id=dst,
                device_id_type=pl.DeviceIdType.LOGICAL,
            )
            pl.semaphore_wait(barrier, 1)
            rdma = pltpu.make_async_remote_copy(
                src_ref=x_ref,
                dst_ref=out_ref,
                send_sem=d_send,
                recv_sem=d_recv,
                device_id=dst,
                device_id_type=pl.DeviceIdType.LOGICAL,
            )
            rdma.start()
            rdma.wait()

        @pl.when(s != 2)
        def _():
            rev = lax.rem(my + N_DEV - s, N_DEV)

            for nbr in (dst, rev):
                pl.semaphore_signal(
                    barrier,
                    inc=1,
                    device_id=nbr,
                    device_id_type=pl.DeviceIdType.LOGICAL,
                )
            pl.semaphore_wait(barrier, 2)

            direct = pltpu.make_async_remote_copy(
                src_ref=x_ref.at[:, pl.ds(0, DIRECT_ROWS), :],
                dst_ref=out_ref.at[:, pl.ds(0, DIRECT_ROWS), :],
                send_sem=d_send,
                recv_sem=d_recv,
                device_id=dst,
                device_id_type=pl.DeviceIdType.LOGICAL,
            )
            direct.start()

            def stage_copy(st, c):
                hbm_rows = pl.ds(DIRECT_ROWS + c * SUB_ROWS, SUB_ROWS)
                vmem_rows = pl.ds(c * SUB_ROWS, SUB_ROWS)
                srcs = [
                    x_ref.at[:, hbm_rows, :],
                    t1.at[:, vmem_rows, :],
                    t2.at[:, vmem_rows, :],
                ]
                dsts = [
                    t1.at[:, vmem_rows, :],
                    t2.at[:, vmem_rows, :],
                    out_ref.at[:, hbm_rows, :],
                ]
                return pltpu.make_async_remote_copy(
                    src_ref=srcs[st],
                    dst_ref=dsts[st],
                    send_sem=r_send.at[st, c],
                    recv_sem=r_recv.at[st, c],
                    device_id=rev,
                    device_id_type=pl.DeviceIdType.LOGICAL,
                )

            stages = [[stage_copy(st, c) for c in range(N_SUB)]
                      for st in range(N_STAGES)]

            for c in range(N_SUB):
                stages[0][c].start()
            for st in range(N_STAGES - 1):
                for c in range(N_SUB):
                    stages[st][c].wait_recv()
                    stages[st + 1][c].start()
            for c in range(N_SUB):
                stages[N_STAGES - 1][c].wait_recv()

            for st in range(N_STAGES):
                for c in range(N_SUB):
                    stages[st][c].wait_send()
            direct.wait()

    return pl.pallas_call(
        body,
        out_shape=jax.ShapeDtypeStruct(x.shape, x.dtype),
        in_specs=[
            pl.BlockSpec(memory_space=pl.ANY),
            pl.BlockSpec(memory_space=pltpu.SMEM),
        ],
        out_specs=pl.BlockSpec(memory_space=pl.ANY),
        scratch_shapes=[
            pltpu.VMEM((1, REV_ROWS, 2048), jnp.float32),
            pltpu.VMEM((1, REV_ROWS, 2048), jnp.float32),
            pltpu.SemaphoreType.DMA,
            pltpu.SemaphoreType.DMA,
            pltpu.SemaphoreType.DMA((N_STAGES, N_SUB)),
            pltpu.SemaphoreType.DMA((N_STAGES, N_SUB)),
        ],
        compiler_params=pltpu.CompilerParams(
            has_side_effects=True,
            collective_id=0,
        ),
    )(x, pi)
